# Initial kernel scaffold; baseline (speedup 1.0000x reference)
import jax
import jax.numpy as jnp
from jax.experimental import pallas as pl


def kernel(x, edge_index, W1, b1, W2, b2):
    raise NotImplementedError("write your pallas kernel here")


# speedup vs baseline: 15.1926x; 15.1926x over previous
import functools

import jax
import jax.numpy as jnp
from jax import lax
from jax.experimental import pallas as pl
from jax.experimental.pallas import tpu as pltpu
from jax.experimental.pallas import tpu_sc as plsc

N = 10000
E = 320000
DIN = 128
HID = 64

NC = 2
NS = 16
NW = NC * NS

NP = 10240
CH = 128
EPT = 10240
EP = EPT * NW
NCH = EPT // CH
RPT = NP // NS

@functools.cache
def _mesh():
    return plsc.VectorSubcoreMesh(
        core_axis_name="c", subcore_axis_name="s",
        num_cores=NC, num_subcores=NS)


def _deg_body(dst_hbm, out_hbm, deg_v, idx_v):
    cid = lax.axis_index("c")
    sid = lax.axis_index("s")
    wid = sid * NC + cid
    zeros16 = jnp.zeros((16,), jnp.float32)

    @pl.loop(0, NP // 16)
    def _(i):
        deg_v[pl.ds(i * 16, 16)] = zeros16

    pltpu.sync_copy(dst_hbm.at[pl.ds(wid * EPT, EPT)], idx_v)
    ones16 = jnp.ones((16,), jnp.float32)

    @pl.loop(0, EPT // 16)
    def _(i):
        idx = idx_v[pl.ds(i * 16, 16)]
        plsc.addupdate_scatter(deg_v, [idx], ones16)

    pltpu.sync_copy(deg_v, out_hbm.at[wid])


@functools.cache
def _deg_call():
    return pl.kernel(
        _deg_body,
        out_type=jax.ShapeDtypeStruct((NW, NP), jnp.float32),
        mesh=_mesh(),
        scratch_types=[
            pltpu.VMEM((NP,), jnp.float32),
            pltpu.VMEM((EPT,), jnp.int32),
        ],
        compiler_params=pltpu.CompilerParams(needs_layout_passes=False),
    )


def _scat_body(g_hbm, src_hbm, dst_hbm, out_hbm, srcv, dstv, rows, accum,
               zbuf, sem):
    cid = lax.axis_index("c")
    sid = lax.axis_index("s")
    wid = sid * NC + cid
    zeros16 = jnp.zeros((16,), jnp.float32)

    @pl.loop(0, 64)
    def _(i):
        @pl.loop(0, HID // 16)
        def _(j):
            zbuf[i, pl.ds(j * 16, 16)] = zeros16

    @pl.loop(0, RPT // 64)
    def _(k):
        pltpu.sync_copy(zbuf, accum.at[pl.ds(sid * RPT + k * 64, 64)])

    pltpu.sync_copy(src_hbm.at[wid], srcv)
    pltpu.sync_copy(dst_hbm.at[wid], dstv)
    plsc.subcore_barrier()

    @pl.loop(0, NCH)
    def _(j):
        pltpu.async_copy(g_hbm.at[srcv.at[j]], rows, sem).wait()
        pltpu.sync_copy(rows, accum.at[dstv.at[j]], add=True)

    plsc.subcore_barrier()
    pltpu.sync_copy(accum.at[pl.ds(sid * RPT, RPT)],
                    out_hbm.at[pl.ds(cid * NP + sid * RPT, RPT)])


@functools.cache
def _scat_call():
    return pl.kernel(
        _scat_body,
        out_type=jax.ShapeDtypeStruct((NC * NP, HID), jnp.float32),
        mesh=_mesh(),
        scratch_types=[
            pltpu.VMEM((NCH, CH), jnp.int32),
            pltpu.VMEM((NCH, CH), jnp.int32),
            pltpu.VMEM((CH, HID), jnp.float32),
            pltpu.VMEM_SHARED((NP, HID), jnp.float32),
            pltpu.VMEM((64, HID), jnp.float32),
            pltpu.SemaphoreType.DMA,
        ],
        compiler_params=pltpu.CompilerParams(use_tc_tiling_on_sc=False),
    )


_RB = 256
_GRID = NP // _RB


def _dense1_body(dp_ref, x_ref, w1_ref, dinv_ref, h1_ref, g1_ref):
    deg = jnp.sum(dp_ref[...], axis=0) + 1.0
    dinv = lax.rsqrt(jnp.maximum(deg, 1e-12))[:, None]
    dinv_ref[...] = dinv
    h = jnp.dot(x_ref[...], w1_ref[...], preferred_element_type=jnp.float32)
    h1_ref[...] = h
    g1_ref[...] = h * dinv


_dense1_call = pl.pallas_call(
    _dense1_body,
    grid=(_GRID,),
    in_specs=[
        pl.BlockSpec((NW, _RB), lambda i: (0, i)),
        pl.BlockSpec((_RB, DIN), lambda i: (i, 0)),
        pl.BlockSpec((DIN, HID), lambda i: (0, 0)),
    ],
    out_specs=[
        pl.BlockSpec((_RB, 1), lambda i: (i, 0)),
        pl.BlockSpec((_RB, HID), lambda i: (i, 0)),
        pl.BlockSpec((_RB, HID), lambda i: (i, 0)),
    ],
    out_shape=[
        jax.ShapeDtypeStruct((NP, 1), jnp.float32),
        jax.ShapeDtypeStruct((NP, HID), jnp.float32),
        jax.ShapeDtypeStruct((NP, HID), jnp.float32),
    ],
)


def _dense2_body(s_ref, h1_ref, dinv_ref, b1_ref, w2_ref, h2_ref, g2_ref):
    dinv = dinv_ref[...]
    s = s_ref[0] + s_ref[1]
    h1f = jnp.maximum(dinv * s + dinv * dinv * h1_ref[...] + b1_ref[...], 0.0)
    h2 = jnp.dot(h1f, w2_ref[...], preferred_element_type=jnp.float32)
    h2_ref[...] = h2
    g2_ref[...] = h2 * dinv


_dense2_call = pl.pallas_call(
    _dense2_body,
    grid=(_GRID,),
    in_specs=[
        pl.BlockSpec((NC, _RB, HID), lambda i: (0, i, 0)),
        pl.BlockSpec((_RB, HID), lambda i: (i, 0)),
        pl.BlockSpec((_RB, 1), lambda i: (i, 0)),
        pl.BlockSpec((1, HID), lambda i: (0, 0)),
        pl.BlockSpec((HID, HID), lambda i: (0, 0)),
    ],
    out_specs=[
        pl.BlockSpec((_RB, HID), lambda i: (i, 0)),
        pl.BlockSpec((_RB, HID), lambda i: (i, 0)),
    ],
    out_shape=[
        jax.ShapeDtypeStruct((NP, HID), jnp.float32),
        jax.ShapeDtypeStruct((NP, HID), jnp.float32),
    ],
)


def _dense3_body(s_ref, h2_ref, dinv_ref, b2_ref, o_ref):
    dinv = dinv_ref[...]
    s = s_ref[0] + s_ref[1]
    o_ref[...] = dinv * s + dinv * dinv * h2_ref[...] + b2_ref[...]


_dense3_call = pl.pallas_call(
    _dense3_body,
    grid=(_GRID,),
    in_specs=[
        pl.BlockSpec((NC, _RB, HID), lambda i: (0, i, 0)),
        pl.BlockSpec((_RB, HID), lambda i: (i, 0)),
        pl.BlockSpec((_RB, 1), lambda i: (i, 0)),
        pl.BlockSpec((1, HID), lambda i: (0, 0)),
    ],
    out_specs=pl.BlockSpec((_RB, HID), lambda i: (i, 0)),
    out_shape=jax.ShapeDtypeStruct((NP, HID), jnp.float32),
)


def kernel(x, edge_index, W1, b1, W2, b2):
    src = edge_index[0]
    dst = edge_index[1]
    pad = EP - E
    src_p = jnp.concatenate([src, jnp.zeros((pad,), src.dtype)])
    dst_p = jnp.concatenate([dst, jnp.full((pad,), NP - 1, dst.dtype)])
    src_r = src_p.reshape(NW, NCH, CH)
    dst_r = dst_p.reshape(NW, NCH, CH)
    x_p = jnp.concatenate([x, jnp.zeros((NP - N, DIN), x.dtype)])

    deg_parts = _deg_call()(dst_p)
    dinv, h1, g1 = _dense1_call(deg_parts, x_p, W1)
    s1 = _scat_call()(g1, src_r, dst_r).reshape(NC, NP, HID)
    h2, g2 = _dense2_call(s1, h1, dinv, b1.reshape(1, HID), W2)
    s2 = _scat_call()(g2, src_r, dst_r).reshape(NC, NP, HID)
    out = _dense3_call(s2, h2, dinv, b2.reshape(1, HID))
    return out[:N]
